# baseline (speedup 1.0000x reference)
import functools

import jax
import jax.numpy as jnp
from jax import lax
from jax.experimental import pallas as pl
from jax.experimental.pallas import tpu as pltpu
from jax.experimental.pallas import tpu_sc as plsc

SEQ = 8192
DIM = 1024
BATCH = 4
NC = 2
NS = 16
NW = NC * NS
ROWS_PER_W = SEQ // NW
R_BLK = 32
BLK_WORDS = R_BLK * DIM
N_BLKS = ROWS_PER_W // R_BLK

_mesh = plsc.VectorSubcoreMesh(core_axis_name="c", subcore_axis_name="s")


@functools.partial(
    pl.kernel,
    mesh=_mesh,
    out_type=jax.ShapeDtypeStruct((BATCH * SEQ * DIM,), jnp.float32),
    scratch_types=[
        pltpu.VMEM((BLK_WORDS,), jnp.float32),
        pltpu.VMEM((BLK_WORDS,), jnp.float32),
    ],
)
def _sc_add(x_hbm, t_hbm, o_hbm, tbuf, xbuf):
    wid = lax.axis_index("s") * NC + lax.axis_index("c")
    row0 = wid * ROWS_PER_W

    def add_body(i, _):
        base = i * 128
        for j in range(8):
            s = base + j * 16
            xbuf[pl.ds(s, 16)] = xbuf[pl.ds(s, 16)] + tbuf[pl.ds(s, 16)]
        return 0

    for blk in range(N_BLKS):
        t_off = (row0 + blk * R_BLK) * DIM
        pltpu.sync_copy(t_hbm.at[pl.ds(t_off, BLK_WORDS)], tbuf)
        for b in range(BATCH):
            x_off = b * SEQ * DIM + t_off
            pltpu.sync_copy(x_hbm.at[pl.ds(x_off, BLK_WORDS)], xbuf)
            lax.fori_loop(0, BLK_WORDS // 128, add_body, 0)
            pltpu.sync_copy(xbuf, o_hbm.at[pl.ds(x_off, BLK_WORDS)])


def kernel(inputs, pos_table):
    out_flat = _sc_add(inputs.reshape(-1), pos_table.reshape(-1))
    return out_flat.reshape(inputs.shape)


# speedup vs baseline: 1.1719x; 1.1719x over previous
import functools

import jax
import jax.numpy as jnp
from jax import lax
from jax.experimental import pallas as pl
from jax.experimental.pallas import tpu as pltpu
from jax.experimental.pallas import tpu_sc as plsc

SEQ = 8192
DIM = 1024
BATCH = 4
NC = 2
NS = 16
NW = NC * NS
ROWS_PER_W = SEQ // NW
R_BLK = 32
BLK_WORDS = R_BLK * DIM
N_BLKS = ROWS_PER_W // R_BLK
N_TASKS = N_BLKS * BATCH

_mesh = plsc.VectorSubcoreMesh(core_axis_name="c", subcore_axis_name="s")


@functools.partial(
    pl.kernel,
    mesh=_mesh,
    out_type=jax.ShapeDtypeStruct((BATCH * SEQ * DIM,), jnp.float32),
    scratch_types=[
        pltpu.VMEM((BLK_WORDS,), jnp.float32),
        pltpu.VMEM((BLK_WORDS,), jnp.float32),
        pltpu.VMEM((BLK_WORDS,), jnp.float32),
        pltpu.SemaphoreType.DMA,
        pltpu.SemaphoreType.DMA,
        pltpu.SemaphoreType.DMA,
        pltpu.SemaphoreType.DMA,
        pltpu.SemaphoreType.DMA,
    ],
)
def _sc_add(x_hbm, t_hbm, o_hbm, tbuf, xb0, xb1, sem_t, sem_l0, sem_l1,
            sem_s0, sem_s1):
    wid = lax.axis_index("s") * NC + lax.axis_index("c")
    row0 = wid * ROWS_PER_W
    xbuf = (xb0, xb1)
    sem_l = (sem_l0, sem_l1)
    sem_s = (sem_s0, sem_s1)

    def x_off(task):
        blk, b = divmod(task, BATCH)
        return b * SEQ * DIM + (row0 + blk * R_BLK) * DIM

    def make_add(buf):
        def add_body(i, _):
            base = i * 128
            for j in range(8):
                s = base + j * 16
                buf[pl.ds(s, 16)] = buf[pl.ds(s, 16)] + tbuf[pl.ds(s, 16)]
            return 0
        return add_body

    load_h = [None] * N_TASKS
    store_h = [None] * N_TASKS
    load_h[0] = pltpu.async_copy(
        x_hbm.at[pl.ds(x_off(0), BLK_WORDS)], xbuf[0], sem_l[0])
    for t in range(N_TASKS):
        blk, b = divmod(t, BATCH)
        cur = t % 2
        nxt = (t + 1) % 2
        if b == 0:
            t_off = (row0 + blk * R_BLK) * DIM
            pltpu.async_copy(
                t_hbm.at[pl.ds(t_off, BLK_WORDS)], tbuf, sem_t).wait()
        if t + 1 < N_TASKS:
            if t >= 1:
                store_h[t - 1].wait()
            load_h[t + 1] = pltpu.async_copy(
                x_hbm.at[pl.ds(x_off(t + 1), BLK_WORDS)], xbuf[nxt],
                sem_l[nxt])
        load_h[t].wait()
        lax.fori_loop(0, BLK_WORDS // 128, make_add(xbuf[cur]), 0)
        store_h[t] = pltpu.async_copy(
            xbuf[cur], o_hbm.at[pl.ds(x_off(t), BLK_WORDS)], sem_s[cur])
    store_h[N_TASKS - 2].wait()
    store_h[N_TASKS - 1].wait()


def kernel(inputs, pos_table):
    out_flat = _sc_add(inputs.reshape(-1), pos_table.reshape(-1))
    return out_flat.reshape(inputs.shape)


# speedup vs baseline: 3.5471x; 3.0269x over previous
import functools

import jax
import jax.numpy as jnp
from jax import lax
from jax.experimental import pallas as pl
from jax.experimental.pallas import tpu as pltpu
from jax.experimental.pallas import tpu_sc as plsc

SEQ = 8192
DIM = 1024
BATCH = 4
NC = 2
NS = 16
NW = NC * NS
ROWS_PER_W = SEQ // NW
R_BLK = 16
N_BLKS = ROWS_PER_W // R_BLK
N_TASKS = N_BLKS * BATCH
NBUF = 4

_mesh = plsc.VectorSubcoreMesh(core_axis_name="c", subcore_axis_name="s")


@functools.partial(
    pl.kernel,
    mesh=_mesh,
    out_type=jax.ShapeDtypeStruct((BATCH, SEQ, DIM), jnp.float32),
    scratch_types=(
        [pltpu.VMEM((R_BLK, DIM), jnp.float32) for _ in range(2)]
        + [pltpu.VMEM((R_BLK, DIM), jnp.float32) for _ in range(NBUF)]
        + [pltpu.SemaphoreType.DMA for _ in range(2 + 2 * NBUF)]
    ),
)
def _sc_add(x_hbm, t_hbm, o_hbm, tb0, tb1, xb0, xb1, xb2, xb3, st0, st1,
            sl0, sl1, sl2, sl3, ss0, ss1, ss2, ss3):
    wid = lax.axis_index("s") * NC + lax.axis_index("c")
    row0 = wid * ROWS_PER_W
    tbuf = (tb0, tb1)
    xbuf = (xb0, xb1, xb2, xb3)
    sem_t = (st0, st1)
    sem_l = (sl0, sl1, sl2, sl3)
    sem_s = (ss0, ss1, ss2, ss3)

    def t_rows(blk):
        return pl.ds(row0 + blk * R_BLK, R_BLK)

    def make_add(xb, tb):
        def add_body(i, _):
            r = lax.shift_right_logical(i, 3)
            c = lax.shift_left(lax.bitwise_and(i, 7), 7)
            for j in range(8):
                sl = pl.ds(pl.multiple_of(c + j * 16, 16), 16)
                xb[r, sl] = xb[r, sl] + tb[r, sl]
            return 0
        return add_body

    tload_h = [None] * N_BLKS
    load_h = [None] * N_TASKS
    store_h = [None] * N_TASKS
    tload_h[0] = pltpu.async_copy(t_hbm.at[t_rows(0)], tbuf[0], sem_t[0])
    load_h[0] = pltpu.async_copy(x_hbm.at[0, t_rows(0), :], xbuf[0], sem_l[0])
    for t in range(N_TASKS):
        blk, b = divmod(t, BATCH)
        slot = t % NBUF
        if b == 0:
            tload_h[blk].wait()
            if blk + 1 < N_BLKS:
                ts = (blk + 1) % 2
                tload_h[blk + 1] = pltpu.async_copy(
                    t_hbm.at[t_rows(blk + 1)], tbuf[ts], sem_t[ts])
        if t + 1 < N_TASKS:
            nslot = (t + 1) % NBUF
            if t + 1 >= NBUF:
                store_h[t + 1 - NBUF].wait()
            nblk, nb = divmod(t + 1, BATCH)
            load_h[t + 1] = pltpu.async_copy(
                x_hbm.at[nb, t_rows(nblk), :], xbuf[nslot], sem_l[nslot])
        load_h[t].wait()
        lax.fori_loop(0, (R_BLK * DIM) // 128,
                      make_add(xbuf[slot], tbuf[blk % 2]), 0)
        store_h[t] = pltpu.async_copy(
            xbuf[slot], o_hbm.at[b, t_rows(blk), :], sem_s[slot])
    for t in range(N_TASKS - NBUF, N_TASKS):
        store_h[t].wait()


def kernel(inputs, pos_table):
    return _sc_add(inputs, pos_table)
